# baseline (speedup 1.0000x reference)
import functools
import math

import jax
import jax.numpy as jnp
from jax.experimental import pallas as pl
from jax.experimental.pallas import tpu as pltpu

_LOG2E = math.log2(math.e)
_LN2 = math.log(2.0)
_CH = 512


def _sp(x):
    a = jax.lax.abs(x)
    e = jnp.exp2(a * (-_LOG2E))
    u = jnp.log2(1.0 + e)
    return jnp.maximum(x, 0.0) + _LN2 * u


def _body(t_ref, x_ref, out_ref, *, br, n, nblocks):
    j = pl.program_id(0)
    t = t_ref[...]
    nfull = n // _CH
    iota = jax.lax.broadcasted_iota(jnp.int32, (br, _CH), 1)

    def chunk(k, acc):
        xc = x_ref[:, pl.ds(k * _CH, _CH)]
        sp = _sp(xc)
        tloc = t - k * _CH
        contrib = sp - jnp.where(iota == tloc, xc, 0.0)
        c = (contrib[:, 0:128] + contrib[:, 128:256]) + (
            contrib[:, 256:384] + contrib[:, 384:512]
        )
        return acc + c

    acc = jax.lax.fori_loop(
        0, nfull, chunk, jnp.zeros((br, 128), jnp.float32)
    )
    rowsum = jnp.sum(acc, axis=1, keepdims=True)

    ntail = n - nfull * _CH
    if ntail:
        xt = x_ref[:, nfull * _CH : n]
        iota_t = nfull * _CH + jax.lax.broadcasted_iota(
            jnp.int32, (br, ntail), 1
        )
        contrib_t = _sp(xt) - jnp.where(iota_t == t, xt, 0.0)
        rowsum = rowsum + jnp.sum(contrib_t, axis=1, keepdims=True)

    good = t != 0
    psum = jnp.sum(jnp.where(good, rowsum, 0.0))
    pcnt = jnp.sum(jnp.where(good, 0.0, 1.0))

    @pl.when(j == 0)
    def _():
        out_ref[0, 0] = 0.0
        out_ref[0, 1] = 0.0

    out_ref[0, 0] += psum
    out_ref[0, 1] += pcnt

    @pl.when(j == nblocks - 1)
    def _():
        out_ref[0, 0] = out_ref[0, 0] / jnp.maximum(out_ref[0, 1], 1.0)


def kernel(input, target):
    m, n = input.shape
    br = 32
    nblocks = m // br
    t = target.astype(jnp.int32).reshape(m, 1)
    out = pl.pallas_call(
        functools.partial(_body, br=br, n=n, nblocks=nblocks),
        grid=(nblocks,),
        in_specs=[
            pl.BlockSpec((br, 1), lambda j: (j, 0)),
            pl.BlockSpec((br, n), lambda j: (j, 0)),
        ],
        out_specs=pl.BlockSpec(
            (1, 2), lambda j: (0, 0), memory_space=pltpu.SMEM
        ),
        out_shape=jax.ShapeDtypeStruct((1, 2), jnp.float32),
        compiler_params=pltpu.CompilerParams(
            dimension_semantics=("arbitrary",)
        ),
    )(t, input)
    return out[0, 0]
